# baseline (device time: 13559 ns/iter reference)
import jax
import jax.numpy as jnp
from jax import lax
from jax.experimental import pallas as pl
from jax.experimental.pallas import tpu as pltpu

N_GLOBAL = 2048
EPS = 1e-5
K = 8


def kernel(x, gamma):
    m, n = x.shape
    rows = m // K
    mh = m // 2
    gamma2d = gamma.reshape(1, n)

    def unpack(packed_inv, mh):
        blk = lax.broadcasted_iota(jnp.int32, (mh, 8), 0) // 128
        sel = (blk == lax.broadcasted_iota(jnp.int32, (mh, 8), 1)).astype(
            jnp.float32
        )
        bcast = lax.dot(sel, packed_inv)
        lane = lax.broadcasted_iota(jnp.int32, (mh, 128), 0) % 128
        mask = lane == lax.broadcasted_iota(jnp.int32, (mh, 128), 1)
        return jnp.sum(jnp.where(mask, bcast, 0.0), axis=1, keepdims=True)

    def body(x_ref, g_ref, o_ref, xv, gv, ov, acc, recv, in_sems, g_sem,
             out_sems, send_sems, recv_sems):
        my_x = lax.axis_index("x")
        my_y = lax.axis_index("y")
        nbr = (my_x, 1 - my_y)

        g_copy = pltpu.make_async_copy(g_ref, gv, g_sem)
        g_copy.start()
        in_copies = []
        for k in range(K):
            cp = pltpu.make_async_copy(
                x_ref.at[pl.ds(k * rows, rows), :],
                xv.at[pl.ds(k * rows, rows), :],
                in_sems.at[k],
            )
            cp.start()
            in_copies.append(cp)

        barrier = pltpu.get_barrier_semaphore()
        pl.semaphore_signal(
            barrier, inc=1, device_id=nbr, device_id_type=pl.DeviceIdType.MESH
        )
        pl.semaphore_wait(barrier, 1)

        def half_rdma(h):
            return pltpu.make_async_remote_copy(
                src_ref=acc.at[pl.ds(h * 8, 8), :],
                dst_ref=recv.at[pl.ds(h * 8, 8), :],
                send_sem=send_sems.at[h],
                recv_sem=recv_sems.at[h],
                device_id=nbr,
                device_id_type=pl.DeviceIdType.MESH,
            )

        rdma = [half_rdma(0), half_rdma(1)]
        for k in range(K):
            in_copies[k].wait()
            xk = xv[pl.ds(k * rows, rows), :]
            acc[pl.ds(2 * k, 2), :] = jnp.sum(xk * xk, axis=1).reshape(2, 128)
            if k == K // 2 - 1:
                rdma[0].start()
        rdma[1].start()

        g_copy.wait()
        g = gv[:, :]
        out_copies = []

        def emit_half(h, inv_h):
            for k in range(h * (K // 2), (h + 1) * (K // 2)):
                r0 = k * rows
                lo = (k % (K // 2)) * rows
                chunk = (xv[pl.ds(r0, rows), :] * g) * inv_h[lo : lo + rows, :]
                ov[pl.ds(r0, rows), :] = chunk.astype(jnp.bfloat16)
                cp = pltpu.make_async_copy(
                    ov.at[pl.ds(r0, rows), :],
                    o_ref.at[pl.ds(r0, rows), :],
                    out_sems.at[k],
                )
                cp.start()
                out_copies.append(cp)

        rdma[0].wait()
        total0 = acc[pl.ds(0, 8), :] + recv[pl.ds(0, 8), :]
        inv0 = unpack(lax.rsqrt(total0 * (1.0 / N_GLOBAL) + EPS), mh)
        emit_half(0, inv0)

        rdma[1].wait()
        total1 = acc[pl.ds(8, 8), :] + recv[pl.ds(8, 8), :]
        inv1 = unpack(lax.rsqrt(total1 * (1.0 / N_GLOBAL) + EPS), mh)
        emit_half(1, inv1)

        for cp in out_copies:
            cp.wait()

    x = pltpu.with_memory_space_constraint(x, pltpu.MemorySpace.HBM)
    gamma2d = pltpu.with_memory_space_constraint(gamma2d, pltpu.MemorySpace.HBM)
    return pl.pallas_call(
        body,
        out_shape=pltpu.MemorySpace.HBM((m, n), jnp.bfloat16),
        in_specs=[
            pl.BlockSpec(memory_space=pltpu.MemorySpace.HBM),
            pl.BlockSpec(memory_space=pltpu.MemorySpace.HBM),
        ],
        out_specs=pl.BlockSpec(memory_space=pltpu.MemorySpace.HBM),
        scratch_shapes=[
            pltpu.VMEM((m, n), jnp.float32),
            pltpu.VMEM((1, n), jnp.float32),
            pltpu.VMEM((m, n), jnp.bfloat16),
            pltpu.VMEM((16, 128), jnp.float32),
            pltpu.VMEM((16, 128), jnp.float32),
            pltpu.SemaphoreType.DMA((K,)),
            pltpu.SemaphoreType.DMA,
            pltpu.SemaphoreType.DMA((K,)),
            pltpu.SemaphoreType.DMA((2,)),
            pltpu.SemaphoreType.DMA((2,)),
        ],
        compiler_params=pltpu.CompilerParams(collective_id=0),
    )(x, gamma2d)


# device time: 9844 ns/iter; 1.3774x vs baseline; 1.3774x over previous
import jax
import jax.numpy as jnp
from jax import lax
from jax.experimental import pallas as pl
from jax.experimental.pallas import tpu as pltpu

N_GLOBAL = 2048
EPS = 1e-5
K = 8


def kernel(x, gamma):
    m, n = x.shape
    rows = m // K
    mh = m // 2
    gamma2d = gamma.reshape(1, n)

    def unpack(packed_inv, mh):
        blk = lax.broadcasted_iota(jnp.int32, (mh, 8), 0) // 128
        sel = (blk == lax.broadcasted_iota(jnp.int32, (mh, 8), 1)).astype(
            jnp.float32
        )
        bcast = lax.dot(sel, packed_inv)
        lane = lax.broadcasted_iota(jnp.int32, (mh, 128), 0) % 128
        mask = lane == lax.broadcasted_iota(jnp.int32, (mh, 128), 1)
        return jnp.sum(jnp.where(mask, bcast, 0.0), axis=1, keepdims=True)

    def body(x_ref, g_ref, o_ref, xv, gv, acc, recv, in_sems, g_sem,
             send_sems, recv_sems):
        my_x = lax.axis_index("x")
        my_y = lax.axis_index("y")
        nbr = (my_x, 1 - my_y)

        g_copy = pltpu.make_async_copy(g_ref, gv, g_sem)
        g_copy.start()
        in_copies = []
        for k in range(K):
            cp = pltpu.make_async_copy(
                x_ref.at[pl.ds(k * rows, rows), :],
                xv.at[pl.ds(k * rows, rows), :],
                in_sems.at[k],
            )
            cp.start()
            in_copies.append(cp)

        barrier = pltpu.get_barrier_semaphore()
        pl.semaphore_signal(
            barrier, inc=1, device_id=nbr, device_id_type=pl.DeviceIdType.MESH
        )
        pl.semaphore_wait(barrier, 1)

        def half_rdma(h):
            return pltpu.make_async_remote_copy(
                src_ref=acc.at[pl.ds(h * 8, 8), :],
                dst_ref=recv.at[pl.ds(h * 8, 8), :],
                send_sem=send_sems.at[h],
                recv_sem=recv_sems.at[h],
                device_id=nbr,
                device_id_type=pl.DeviceIdType.MESH,
            )

        rdma = [half_rdma(0), half_rdma(1)]
        for k in range(K):
            in_copies[k].wait()
            xk = xv[pl.ds(k * rows, rows), :]
            acc[pl.ds(2 * k, 2), :] = jnp.sum(xk * xk, axis=1).reshape(2, 128)
            if k == K // 2 - 1:
                rdma[0].start()
        rdma[1].start()

        g_copy.wait()
        g = gv[:, :]

        def emit_half(h, inv_h):
            for k in range(h * (K // 2), (h + 1) * (K // 2)):
                r0 = k * rows
                lo = (k % (K // 2)) * rows
                chunk = (xv[pl.ds(r0, rows), :] * g) * inv_h[lo : lo + rows, :]
                o_ref[pl.ds(r0, rows), :] = chunk.astype(jnp.bfloat16)

        rdma[0].wait()
        total0 = acc[pl.ds(0, 8), :] + recv[pl.ds(0, 8), :]
        inv0 = unpack(lax.rsqrt(total0 * (1.0 / N_GLOBAL) + EPS), mh)
        emit_half(0, inv0)

        rdma[1].wait()
        total1 = acc[pl.ds(8, 8), :] + recv[pl.ds(8, 8), :]
        inv1 = unpack(lax.rsqrt(total1 * (1.0 / N_GLOBAL) + EPS), mh)
        emit_half(1, inv1)

    x = pltpu.with_memory_space_constraint(x, pltpu.MemorySpace.HBM)
    gamma2d = pltpu.with_memory_space_constraint(gamma2d, pltpu.MemorySpace.HBM)
    return pl.pallas_call(
        body,
        out_shape=jax.ShapeDtypeStruct((m, n), jnp.bfloat16),
        in_specs=[
            pl.BlockSpec(memory_space=pltpu.MemorySpace.HBM),
            pl.BlockSpec(memory_space=pltpu.MemorySpace.HBM),
        ],
        out_specs=pl.BlockSpec(memory_space=pltpu.VMEM),
        scratch_shapes=[
            pltpu.VMEM((m, n), jnp.float32),
            pltpu.VMEM((1, n), jnp.float32),
            pltpu.VMEM((16, 128), jnp.float32),
            pltpu.VMEM((16, 128), jnp.float32),
            pltpu.SemaphoreType.DMA((K,)),
            pltpu.SemaphoreType.DMA,
            pltpu.SemaphoreType.DMA((2,)),
            pltpu.SemaphoreType.DMA((2,)),
        ],
        compiler_params=pltpu.CompilerParams(collective_id=0),
    )(x, gamma2d)


# device time: 9735 ns/iter; 1.3928x vs baseline; 1.0112x over previous
import jax
import jax.numpy as jnp
from jax import lax
from jax.experimental import pallas as pl
from jax.experimental.pallas import tpu as pltpu

N_GLOBAL = 2048
EPS = 1e-5
K = 8


def kernel(x, gamma):
    m, n = x.shape
    rows = m // K
    mh = m // 2
    gamma2d = gamma.reshape(1, n)

    def unpack(packed_inv, nr):
        nb = nr // 128
        blk = lax.broadcasted_iota(jnp.int32, (nr, nb), 0) // 128
        sel = (blk == lax.broadcasted_iota(jnp.int32, (nr, nb), 1)).astype(
            jnp.float32
        )
        bcast = lax.dot(sel, packed_inv)
        lane = lax.broadcasted_iota(jnp.int32, (nr, 128), 0) % 128
        mask = lane == lax.broadcasted_iota(jnp.int32, (nr, 128), 1)
        return jnp.sum(jnp.where(mask, bcast, 0.0), axis=1, keepdims=True)

    def body(x_ref, g_ref, o_ref, xv, gv, acc, recv, in_sems, g_sem,
             send_sems, recv_sems):
        my_x = lax.axis_index("x")
        my_y = lax.axis_index("y")
        nbr = (my_x, 1 - my_y)

        g_copy = pltpu.make_async_copy(g_ref, gv, g_sem)
        g_copy.start()
        in_copies = []
        for k in range(K):
            cp = pltpu.make_async_copy(
                x_ref.at[pl.ds(k * rows, rows), :],
                xv.at[pl.ds(k * rows, rows), :],
                in_sems.at[k],
            )
            cp.start()
            in_copies.append(cp)

        barrier = pltpu.get_barrier_semaphore()
        pl.semaphore_signal(
            barrier, inc=1, device_id=nbr, device_id_type=pl.DeviceIdType.MESH
        )
        pl.semaphore_wait(barrier, 1)

        pk = rows // 128

        def chunk_rdma(k):
            return pltpu.make_async_remote_copy(
                src_ref=acc.at[pl.ds(k * pk, pk), :],
                dst_ref=recv.at[pl.ds(k * pk, pk), :],
                send_sem=send_sems.at[k],
                recv_sem=recv_sems.at[k],
                device_id=nbr,
                device_id_type=pl.DeviceIdType.MESH,
            )

        rdmas = [chunk_rdma(k) for k in range(K)]
        for k in range(K):
            in_copies[k].wait()
            xk = xv[pl.ds(k * rows, rows), :]
            acc[pl.ds(k * pk, pk), :] = jnp.sum(xk * xk, axis=1).reshape(
                pk, 128
            )
            rdmas[k].start()

        g_copy.wait()
        g = gv[:, :]

        for k in range(K):
            rdmas[k].wait()
            total = acc[pl.ds(k * pk, pk), :] + recv[pl.ds(k * pk, pk), :]
            inv = unpack(lax.rsqrt(total * (1.0 / N_GLOBAL) + EPS), rows)
            r0 = k * rows
            chunk = (xv[pl.ds(r0, rows), :] * g) * inv
            o_ref[pl.ds(r0, rows), :] = chunk.astype(jnp.bfloat16)

    x = pltpu.with_memory_space_constraint(x, pltpu.MemorySpace.HBM)
    gamma2d = pltpu.with_memory_space_constraint(gamma2d, pltpu.MemorySpace.HBM)
    return pl.pallas_call(
        body,
        out_shape=jax.ShapeDtypeStruct((m, n), jnp.bfloat16),
        in_specs=[
            pl.BlockSpec(memory_space=pltpu.MemorySpace.HBM),
            pl.BlockSpec(memory_space=pltpu.MemorySpace.HBM),
        ],
        out_specs=pl.BlockSpec(memory_space=pltpu.VMEM),
        scratch_shapes=[
            pltpu.VMEM((m, n), jnp.float32),
            pltpu.VMEM((1, n), jnp.float32),
            pltpu.VMEM((16, 128), jnp.float32),
            pltpu.VMEM((16, 128), jnp.float32),
            pltpu.SemaphoreType.DMA((K,)),
            pltpu.SemaphoreType.DMA,
            pltpu.SemaphoreType.DMA((K,)),
            pltpu.SemaphoreType.DMA((K,)),
        ],
        compiler_params=pltpu.CompilerParams(collective_id=0),
    )(x, gamma2d)
